# Initial kernel scaffold; baseline (speedup 1.0000x reference)
import jax
import jax.numpy as jnp
from jax.experimental import pallas as pl


def kernel(Z, Y):
    raise NotImplementedError("write your pallas kernel here")


# speedup vs baseline: 5.4531x; 5.4531x over previous
import jax
import jax.numpy as jnp
from jax.experimental import pallas as pl
from jax.experimental.pallas import tpu as pltpu

_B = 4096
_K = 1024
_BM = 512
_BN = 512
_JT = _B // _BM
_IT = _B // _BN


def _body(y_ref, z_ref, t1_ref, t10_ref, d_ref, cnt_ref):
    j = pl.program_id(0)
    s = pl.program_id(1)
    i_blk = jax.lax.rem(j + s, _IT)

    y = y_ref[...]
    z = z_ref[...]

    dots = jax.lax.dot_general(
        y, z, (((1,), (1,)), ((), ())), preferred_element_type=jnp.float32)

    ones = jnp.ones((1, _K), dtype=jnp.float32)
    ny2 = jax.lax.dot_general(
        y * y, ones, (((1,), (1,)), ((), ())),
        preferred_element_type=jnp.float32)
    nx2 = jax.lax.dot_general(
        ones, z * z, (((1,), (1,)), ((), ())),
        preferred_element_type=jnp.float32)
    denom = jnp.maximum(jnp.sqrt(ny2) * jnp.sqrt(nx2), 1e-8)
    sim = dots / denom

    rows = jax.lax.broadcasted_iota(jnp.int32, (_BM, _BN), 0)
    cols = jax.lax.broadcasted_iota(jnp.int32, (_BM, _BN), 1)

    @pl.when(s == 0)
    def _():
        d_ref[...] = jnp.sum(
            jnp.where(rows == cols, sim, 0.0), axis=1, keepdims=True)

    d = d_ref[...]
    g_j = j * _BM + rows
    g_i = i_blk * _BN + cols
    beats = (sim > d) | ((sim == d) & (g_i < g_j))
    c = jnp.sum(beats.astype(jnp.int32), axis=1, keepdims=True)

    @pl.when(s == 0)
    def _():
        cnt_ref[...] = c

    @pl.when(s > 0)
    def _():
        cnt_ref[...] = cnt_ref[...] + c

    @pl.when((j == 0) & (s == 0))
    def _():
        t1_ref[...] = jnp.zeros_like(t1_ref)
        t10_ref[...] = jnp.zeros_like(t10_ref)

    @pl.when(s == _IT - 1)
    def _():
        cnt = cnt_ref[...]
        t1_ref[...] += jnp.sum(
            (cnt == 0).astype(jnp.float32), keepdims=True) * (1.0 / _B)
        t10_ref[...] += jnp.sum(
            (cnt < 10).astype(jnp.float32), keepdims=True) * (1.0 / _B)


def kernel(Z, Y):
    t1, t10 = pl.pallas_call(
        _body,
        grid=(_JT, _IT),
        in_specs=[
            pl.BlockSpec((_BM, _K), lambda j, s: (j, 0)),
            pl.BlockSpec((_BN, _K),
                         lambda j, s: (jax.lax.rem(j + s, _IT), 0)),
        ],
        out_specs=[
            pl.BlockSpec((1, 1), lambda j, s: (0, 0)),
            pl.BlockSpec((1, 1), lambda j, s: (0, 0)),
        ],
        out_shape=[
            jax.ShapeDtypeStruct((1, 1), jnp.float32),
            jax.ShapeDtypeStruct((1, 1), jnp.float32),
        ],
        scratch_shapes=[
            pltpu.VMEM((_BM, 1), jnp.float32),
            pltpu.VMEM((_BM, 1), jnp.int32),
        ],
        compiler_params=pltpu.CompilerParams(
            dimension_semantics=("arbitrary", "arbitrary")),
    )(Y, Z)
    return (t1[0, 0], t10[0, 0])


# speedup vs baseline: 7.3293x; 1.3441x over previous
import jax
import jax.numpy as jnp
from jax.experimental import pallas as pl
from jax.experimental.pallas import tpu as pltpu

_B = 4096
_K = 1024
_BM = 1024
_BN = 1024
_JT = _B // _BM
_IT = _B // _BN


def _body(y_ref, z_ref, t1_ref, t10_ref, d_ref, cnt_ref):
    j = pl.program_id(0)
    s = pl.program_id(1)
    i_blk = jax.lax.rem(j + s, _IT)

    y = y_ref[...]
    z = z_ref[...]

    dots = jax.lax.dot_general(
        y, z, (((1,), (1,)), ((), ())), preferred_element_type=jnp.float32)

    ones = jnp.ones((1, _K), dtype=jnp.float32)
    ny2 = jax.lax.dot_general(
        y * y, ones, (((1,), (1,)), ((), ())),
        preferred_element_type=jnp.float32)
    nx2 = jax.lax.dot_general(
        ones, z * z, (((1,), (1,)), ((), ())),
        preferred_element_type=jnp.float32)
    denom = jnp.maximum(jnp.sqrt(ny2) * jnp.sqrt(nx2), 1e-8)
    sim = dots / denom

    rows = jax.lax.broadcasted_iota(jnp.int32, (_BM, _BN), 0)
    cols = jax.lax.broadcasted_iota(jnp.int32, (_BM, _BN), 1)

    @pl.when(s == 0)
    def _():
        d_ref[...] = jnp.sum(
            jnp.where(rows == cols, sim, 0.0), axis=1, keepdims=True)

    d = d_ref[...]
    g_j = j * _BM + rows
    g_i = i_blk * _BN + cols
    beats = (sim > d) | ((sim == d) & (g_i < g_j))
    c = jnp.sum(beats.astype(jnp.int32), axis=1, keepdims=True)

    @pl.when(s == 0)
    def _():
        cnt_ref[...] = c

    @pl.when(s > 0)
    def _():
        cnt_ref[...] = cnt_ref[...] + c

    @pl.when((j == 0) & (s == 0))
    def _():
        t1_ref[...] = jnp.zeros_like(t1_ref)
        t10_ref[...] = jnp.zeros_like(t10_ref)

    @pl.when(s == _IT - 1)
    def _():
        cnt = cnt_ref[...]
        t1_ref[...] += jnp.sum(
            (cnt == 0).astype(jnp.float32), keepdims=True) * (1.0 / _B)
        t10_ref[...] += jnp.sum(
            (cnt < 10).astype(jnp.float32), keepdims=True) * (1.0 / _B)


def kernel(Z, Y):
    t1, t10 = pl.pallas_call(
        _body,
        grid=(_JT, _IT),
        in_specs=[
            pl.BlockSpec((_BM, _K), lambda j, s: (j, 0)),
            pl.BlockSpec((_BN, _K),
                         lambda j, s: (jax.lax.rem(j + s, _IT), 0)),
        ],
        out_specs=[
            pl.BlockSpec((1, 1), lambda j, s: (0, 0)),
            pl.BlockSpec((1, 1), lambda j, s: (0, 0)),
        ],
        out_shape=[
            jax.ShapeDtypeStruct((1, 1), jnp.float32),
            jax.ShapeDtypeStruct((1, 1), jnp.float32),
        ],
        scratch_shapes=[
            pltpu.VMEM((_BM, 1), jnp.float32),
            pltpu.VMEM((_BM, 1), jnp.int32),
        ],
        compiler_params=pltpu.CompilerParams(
            dimension_semantics=("arbitrary", "arbitrary")),
    )(Y, Z)
    return (t1[0, 0], t10[0, 0])


# speedup vs baseline: 9.1060x; 1.2424x over previous
import jax
import jax.numpy as jnp
from jax.experimental import pallas as pl
from jax.experimental.pallas import tpu as pltpu

_B = 4096
_K = 1024
_BM = 1024
_BN = 1024
_JT = _B // _BM
_IT = _B // _BN


def _body(y_ref, z_ref, t1_ref, t10_ref, d_ref, ny_ref, nx_ref, acc_ref):
    j = pl.program_id(0)
    s = pl.program_id(1)
    i_blk = jax.lax.rem(j + s, _IT)

    y = y_ref[...]
    z = z_ref[...]

    dots = jax.lax.dot_general(
        y, z, (((1,), (1,)), ((), ())), preferred_element_type=jnp.float32)

    ones_row = jnp.ones((1, _K), dtype=jnp.float32)
    ones_col = jnp.ones((_BN, 1), dtype=jnp.float32)

    @pl.when(s == 0)
    def _():
        ny2 = jax.lax.dot_general(
            y * y, ones_row, (((1,), (1,)), ((), ())),
            preferred_element_type=jnp.float32)
        ny_ref[...] = jnp.sqrt(ny2)

    @pl.when(j == 0)
    def _():
        nx2 = jax.lax.dot_general(
            ones_row, z * z, (((1,), (1,)), ((), ())),
            preferred_element_type=jnp.float32)
        nx_ref[i_blk] = jnp.sqrt(nx2)

    denom = jnp.maximum(ny_ref[...] * nx_ref[i_blk], 1e-8)
    sim = dots / denom

    @pl.when(s == 0)
    def _():
        rows = jax.lax.broadcasted_iota(jnp.int32, (_BM, _BN), 0)
        cols = jax.lax.broadcasted_iota(jnp.int32, (_BM, _BN), 1)
        masked = jnp.where(rows == cols, sim, 0.0)
        d_ref[...] = jax.lax.dot_general(
            masked, ones_col, (((1,), (0,)), ((), ())),
            preferred_element_type=jnp.float32)
        d0 = d_ref[...]
        beats = (sim > d0) | ((sim == d0) & (cols < rows))
        acc_ref[...] = beats.astype(jnp.float32)

    d = d_ref[...]

    @pl.when((s > 0) & (j + s < _IT))
    def _():
        acc_ref[...] += (sim > d).astype(jnp.float32)

    @pl.when(j + s >= _IT)
    def _():
        acc_ref[...] += (sim >= d).astype(jnp.float32)

    @pl.when((j == 0) & (s == 0))
    def _():
        t1_ref[...] = jnp.zeros_like(t1_ref)
        t10_ref[...] = jnp.zeros_like(t10_ref)

    @pl.when(s == _IT - 1)
    def _():
        cnt = jax.lax.dot_general(
            acc_ref[...], ones_col, (((1,), (0,)), ((), ())),
            preferred_element_type=jnp.float32)
        t1_ref[...] += jnp.sum(
            (cnt == 0.0).astype(jnp.float32), keepdims=True) * (1.0 / _B)
        t10_ref[...] += jnp.sum(
            (cnt < 10.0).astype(jnp.float32), keepdims=True) * (1.0 / _B)


def kernel(Z, Y):
    t1, t10 = pl.pallas_call(
        _body,
        grid=(_JT, _IT),
        in_specs=[
            pl.BlockSpec((_BM, _K), lambda j, s: (j, 0)),
            pl.BlockSpec((_BN, _K),
                         lambda j, s: (jax.lax.rem(j + s, _IT), 0)),
        ],
        out_specs=[
            pl.BlockSpec((1, 1), lambda j, s: (0, 0)),
            pl.BlockSpec((1, 1), lambda j, s: (0, 0)),
        ],
        out_shape=[
            jax.ShapeDtypeStruct((1, 1), jnp.float32),
            jax.ShapeDtypeStruct((1, 1), jnp.float32),
        ],
        scratch_shapes=[
            pltpu.VMEM((_BM, 1), jnp.float32),
            pltpu.VMEM((_BM, 1), jnp.float32),
            pltpu.VMEM((_IT, 1, _BN), jnp.float32),
            pltpu.VMEM((_BM, _BN), jnp.float32),
        ],
        compiler_params=pltpu.CompilerParams(
            dimension_semantics=("arbitrary", "arbitrary")),
    )(Y, Z)
    return (t1[0, 0], t10[0, 0])


# speedup vs baseline: 9.3626x; 1.0282x over previous
import jax
import jax.numpy as jnp
from jax.experimental import pallas as pl
from jax.experimental.pallas import tpu as pltpu

_B = 4096
_K = 1024
_BM = 1024
_BN = 1024
_JT = _B // _BM
_IT = _B // _BN


def _body(y_ref, z_ref, t1_ref, t10_ref, d_ref, ny_ref, nx_ref, acc_ref):
    j = pl.program_id(0)
    s = pl.program_id(1)
    i_blk = jax.lax.rem(j + s, _IT)

    y = y_ref[...]
    z = z_ref[...]

    dots = jax.lax.dot_general(
        y, z, (((1,), (1,)), ((), ())), preferred_element_type=jnp.float32)

    ones_row = jnp.ones((1, _K), dtype=jnp.float32)
    ones_col = jnp.ones((_BN, 1), dtype=jnp.float32)

    @pl.when(s == 0)
    def _():
        ny2 = jax.lax.dot_general(
            y * y, ones_row, (((1,), (1,)), ((), ())),
            preferred_element_type=jnp.float32)
        ny_ref[...] = jnp.sqrt(ny2)

    @pl.when(j == 0)
    def _():
        nx2 = jax.lax.dot_general(
            ones_row, z * z, (((1,), (1,)), ((), ())),
            preferred_element_type=jnp.float32)
        nx_ref[i_blk] = jnp.sqrt(nx2)

    denom = jnp.maximum(ny_ref[...] * nx_ref[i_blk], 1e-8)
    sim = dots / denom

    @pl.when(s == 0)
    def _():
        rows = jax.lax.broadcasted_iota(jnp.int32, (_BM, _BN), 0)
        cols = jax.lax.broadcasted_iota(jnp.int32, (_BM, _BN), 1)
        masked = jnp.where(rows == cols, sim, 0.0)
        d_ref[...] = jnp.sum(masked, axis=1, keepdims=True)
        d0 = d_ref[...]
        beats = (sim > d0) | ((sim == d0) & (cols < rows))
        acc_ref[...] = beats.astype(jnp.float32)

    d = d_ref[...]

    @pl.when((s > 0) & (j + s < _IT))
    def _():
        acc_ref[...] += (sim > d).astype(jnp.float32)

    @pl.when(j + s >= _IT)
    def _():
        acc_ref[...] += (sim >= d).astype(jnp.float32)

    @pl.when((j == 0) & (s == 0))
    def _():
        t1_ref[...] = jnp.zeros_like(t1_ref)
        t10_ref[...] = jnp.zeros_like(t10_ref)

    @pl.when(s == _IT - 1)
    def _():
        cnt = jax.lax.dot_general(
            acc_ref[...], ones_col, (((1,), (0,)), ((), ())),
            preferred_element_type=jnp.float32)
        t1_ref[...] += jnp.sum(
            (cnt == 0.0).astype(jnp.float32), keepdims=True) * (1.0 / _B)
        t10_ref[...] += jnp.sum(
            (cnt < 10.0).astype(jnp.float32), keepdims=True) * (1.0 / _B)


def kernel(Z, Y):
    t1, t10 = pl.pallas_call(
        _body,
        grid=(_JT, _IT),
        in_specs=[
            pl.BlockSpec((_BM, _K), lambda j, s: (j, 0)),
            pl.BlockSpec((_BN, _K),
                         lambda j, s: (jax.lax.rem(j + s, _IT), 0)),
        ],
        out_specs=[
            pl.BlockSpec((1, 1), lambda j, s: (0, 0)),
            pl.BlockSpec((1, 1), lambda j, s: (0, 0)),
        ],
        out_shape=[
            jax.ShapeDtypeStruct((1, 1), jnp.float32),
            jax.ShapeDtypeStruct((1, 1), jnp.float32),
        ],
        scratch_shapes=[
            pltpu.VMEM((_BM, 1), jnp.float32),
            pltpu.VMEM((_BM, 1), jnp.float32),
            pltpu.VMEM((_IT, 1, _BN), jnp.float32),
            pltpu.VMEM((_BM, _BN), jnp.float32),
        ],
        compiler_params=pltpu.CompilerParams(
            dimension_semantics=("arbitrary", "arbitrary")),
    )(Y, Z)
    return (t1[0, 0], t10[0, 0])
